# baseline (speedup 1.0000x reference)
import functools

import jax
import jax.numpy as jnp
from jax import lax
from jax.experimental import pallas as pl
from jax.experimental.pallas import tpu as pltpu
from jax.experimental.pallas import tpu_sc as plsc

_F32 = jnp.float32
_NC = 2
_NS = 16
_NW = _NC * _NS
_CH = 128
_SC_PARAMS = pltpu.CompilerParams(use_tc_tiling_on_sc=False)


def _sc_mesh():
    return plsc.VectorSubcoreMesh(core_axis_name="c", subcore_axis_name="s",
                                  num_cores=_NC, num_subcores=_NS)


def _sc_segsum(g, src_g, dst_g, zeros64, zeros16, ones128, with_cnt):
    NP, H = g.shape
    ngr, G, _ = src_g.shape
    nsr = NP // _NS

    stage = False
    out_type = [jax.ShapeDtypeStruct((2 * NP, H), _F32)]
    scratch = [
        pltpu.VMEM((G, _CH), jnp.int32),
        pltpu.VMEM((G, _CH), jnp.int32),
        pltpu.VMEM((G * _CH, H), _F32),
        pltpu.VMEM_SHARED((NP, H), _F32),
        pltpu.SemaphoreType.DMA,
    ]
    if stage:
        scratch.append(pltpu.VMEM_SHARED((NP, H), _F32))
    if with_cnt:
        out_type.append(jax.ShapeDtypeStruct((2 * NP, 16), _F32))
        scratch += [
            pltpu.VMEM((_CH, 16), _F32),
            pltpu.VMEM_SHARED((NP, 16), _F32),
        ]

    @functools.partial(pl.kernel, out_type=out_type, mesh=_sc_mesh(),
                       scratch_types=scratch, compiler_params=_SC_PARAMS)
    def k(*refs):
        if with_cnt:
            (g_hbm, src_hbm, dst_hbm, z64_hbm, z16_hbm, ones_hbm,
             p_hbm, cnt_hbm, srcv, dstv, rows, acc, sem, onesv, cacc) = refs
            gtab = g_hbm
        elif stage:
            (g_hbm, src_hbm, dst_hbm, z64_hbm,
             p_hbm, srcv, dstv, rows, acc, sem, gsh) = refs
            gtab = gsh
        else:
            (g_hbm, src_hbm, dst_hbm, z64_hbm,
             p_hbm, srcv, dstv, rows, acc, sem) = refs
            gtab = g_hbm
        c = lax.axis_index("c")
        s = lax.axis_index("s")
        wid = s * _NC + c
        pltpu.sync_copy(z64_hbm.at[pl.ds(s * nsr, nsr)],
                        acc.at[pl.ds(s * nsr, nsr)])
        if stage:
            pltpu.sync_copy(g_hbm.at[pl.ds(s * nsr, nsr)],
                            gsh.at[pl.ds(s * nsr, nsr)])
        if with_cnt:
            pltpu.sync_copy(z16_hbm.at[pl.ds(s * nsr, nsr)],
                            cacc.at[pl.ds(s * nsr, nsr)])
            pltpu.sync_copy(ones_hbm, onesv)
        plsc.subcore_barrier()

        @pl.loop(wid, ngr, step=_NW)
        def _(grp):
            pltpu.sync_copy(src_hbm.at[grp], srcv)
            pltpu.sync_copy(dst_hbm.at[grp], dstv)
            gathers = [
                pltpu.async_copy(gtab.at[srcv.at[j]],
                                 rows.at[pl.ds(j * _CH, _CH)], sem)
                for j in range(G)
            ]
            for h in gathers:
                h.wait()
            adds = [
                pltpu.async_copy(rows.at[pl.ds(j * _CH, _CH)],
                                 acc.at[dstv.at[j]], sem, add=True)
                for j in range(G)
            ]
            if with_cnt:
                adds += [
                    pltpu.async_copy(onesv, cacc.at[dstv.at[j]], sem,
                                     add=True)
                    for j in range(G)
                ]
            for h in adds:
                h.wait()

        plsc.subcore_barrier()
        pltpu.sync_copy(acc.at[pl.ds(s * nsr, nsr)],
                        p_hbm.at[pl.ds(c * NP + s * nsr, nsr)])
        if with_cnt:
            pltpu.sync_copy(cacc.at[pl.ds(s * nsr, nsr)],
                            cnt_hbm.at[pl.ds(c * NP + s * nsr, nsr)])

    if with_cnt:
        return k(g, src_g, dst_g, zeros64, zeros16, ones128)
    return k(g, src_g, dst_g, zeros64)


def _sc_edge_gather(AB2, SD):
    _, NP, H = AB2.shape
    _, ngr, G, _ = SD.shape
    E = ngr * G * _CH
    slab = G * _CH

    @functools.partial(
        pl.kernel,
        out_type=jax.ShapeDtypeStruct((E, 2 * H), _F32),
        mesh=_sc_mesh(),
        compiler_params=_SC_PARAMS,
        scratch_types=[
            pltpu.VMEM((G, _CH), jnp.int32),
            pltpu.VMEM((slab, H), _F32),
            pltpu.VMEM_SHARED((NP, H), _F32),
            pltpu.SemaphoreType.DMA,
        ],
    )
    def k(ab_hbm, sd_hbm, t_hbm, idxv, rows, tsh, sem):
        c = lax.axis_index("c")
        s = lax.axis_index("s")
        nsr = NP // _NS
        pltpu.sync_copy(ab_hbm.at[c, pl.ds(s * nsr, nsr)],
                        tsh.at[pl.ds(s * nsr, nsr)])
        plsc.subcore_barrier()

        @pl.loop(s, ngr, step=_NS)
        def _(grp):
            pltpu.sync_copy(sd_hbm.at[c, grp], idxv)
            gathers = [
                pltpu.async_copy(tsh.at[idxv.at[j]],
                                 rows.at[pl.ds(j * _CH, _CH)], sem)
                for j in range(G)
            ]
            for h in gathers:
                h.wait()
            pltpu.sync_copy(rows, t_hbm.at[pl.ds(grp * slab, slab),
                                           pl.ds(c * H, H)])

    return k(AB2, SD)


def _tc_stage1(x, Wn, bn, Wl1, Wr1, NP):
    N, D = x.shape
    H = Wn.shape[1]
    BN = 2000

    def body(x_ref, wn_ref, bn_ref, wl_ref, wr_ref, g_ref, r_ref):
        h0 = jnp.maximum(x_ref[...] @ wn_ref[...] + bn_ref[...], 0.0)
        g_ref[...] = h0 @ wl_ref[...]
        r_ref[...] = h0 @ wr_ref[...]

    return pl.pallas_call(
        body,
        grid=(N // BN,),
        in_specs=[
            pl.BlockSpec((BN, D), lambda i: (i, 0)),
            pl.BlockSpec((D, H), lambda i: (0, 0)),
            pl.BlockSpec((1, H), lambda i: (0, 0)),
            pl.BlockSpec((H, H), lambda i: (0, 0)),
            pl.BlockSpec((H, H), lambda i: (0, 0)),
        ],
        out_specs=[pl.BlockSpec((BN, H), lambda i: (i, 0))] * 2,
        out_shape=[jax.ShapeDtypeStruct((NP, H), _F32)] * 2,
    )(x, Wn, bn, Wl1, Wr1)


def _tc_conv_combine(P, CNT, r, bl, Wl, Wr):
    twoNP, H = P.shape
    NP = twoNP // 2
    BN = NP // 8
    NB = NP // BN

    def body(p0, p1, c0r, c1r, r_ref, bl_ref, wl_ref, wr_ref, g_ref, r2_ref):
        S = p0[...] + p1[...]
        cnt = c0r[...][:, 0:1] + c1r[...][:, 0:1]
        h = jnp.maximum(S / jnp.maximum(cnt, 1.0) + bl_ref[...] + r_ref[...],
                        0.0)
        g_ref[...] = h @ wl_ref[...]
        r2_ref[...] = h @ wr_ref[...]

    return pl.pallas_call(
        body,
        grid=(NB,),
        in_specs=[
            pl.BlockSpec((BN, H), lambda i: (i, 0)),
            pl.BlockSpec((BN, H), lambda i: (i + NB, 0)),
            pl.BlockSpec((BN, 16), lambda i: (i, 0)),
            pl.BlockSpec((BN, 16), lambda i: (i + NB, 0)),
            pl.BlockSpec((BN, H), lambda i: (i, 0)),
            pl.BlockSpec((1, H), lambda i: (0, 0)),
            pl.BlockSpec((H, H), lambda i: (0, 0)),
            pl.BlockSpec((H, H), lambda i: (0, 0)),
        ],
        out_specs=[pl.BlockSpec((BN, H), lambda i: (i, 0))] * 2,
        out_shape=[jax.ShapeDtypeStruct((NP, H), _F32)] * 2,
    )(P, P, CNT, CNT, r, bl, Wl, Wr)


def _tc_score(T, ea, Wf, cvec, vv, c0):
    E, H2 = T.shape
    H = H2 // 2
    De = ea.shape[1]
    BE = 512
    NB = E // BE

    def body(t_ref, ea_ref, wf_ref, cv_ref, v_ref, c0_ref, o_ref):
        tt = t_ref[...]
        t = tt[:, :H] + tt[:, H:] + ea_ref[...] @ wf_ref[...] + cv_ref[...]
        o_ref[...] = (jnp.sum(jnp.maximum(t, 0.0) * v_ref[...], axis=1)
                      + c0_ref[0, 0])

    return pl.pallas_call(
        body,
        grid=(NB,),
        in_specs=[
            pl.BlockSpec((BE, 2 * H), lambda i: (i, 0)),
            pl.BlockSpec((BE, De), lambda i: (i, 0)),
            pl.BlockSpec((De, H), lambda i: (0, 0)),
            pl.BlockSpec((1, H), lambda i: (0, 0)),
            pl.BlockSpec((1, H), lambda i: (0, 0)),
            pl.BlockSpec((1, 1), lambda i: (0, 0)),
        ],
        out_specs=pl.BlockSpec((BE,), lambda i: (i,)),
        out_shape=jax.ShapeDtypeStruct((E,), _F32),
    )(T, ea, Wf, cvec, vv, c0)


def kernel(x, edge_index, edge_attr, Wn, bn, Wl1, bl1, Wr1, Wl2, bl2, Wr2,
           We, be, W1, b1, W2, b2, Ws, bs):
    N = x.shape[0]
    H = Wn.shape[1]
    NP = ((N + 127) // 128) * 128

    G1, G2, GE = 5, 10, 5
    src = edge_index[0]
    dst = edge_index[1]
    src_g1 = src.reshape(-1, G1, _CH)
    dst_g1 = dst.reshape(-1, G1, _CH)
    src_g2 = src.reshape(-1, G2, _CH)
    dst_g2 = dst.reshape(-1, G2, _CH)
    SD = jnp.stack([src, dst]).reshape(2, -1, GE, _CH)
    zeros64 = jnp.zeros((NP, H), _F32)
    zeros16 = jnp.zeros((NP, 16), _F32)
    ones128 = jnp.ones((_CH, 16), _F32)

    W1a, W1b, W1c = W1[:H], W1[H:2 * H], W1[2 * H:]
    Wf = We @ W1c
    cvec = (be @ W1c + b1).reshape(1, H)
    vv = W2 @ Ws
    c0 = (b2 @ Ws + bs).reshape(1, 1)

    g1, r1 = _tc_stage1(x, Wn, bn.reshape(1, H), Wl1, Wr1, NP)
    P1, CNT = _sc_segsum(g1, src_g1, dst_g1, zeros64, zeros16, ones128,
                         with_cnt=True)
    g2, r2 = _tc_conv_combine(P1, CNT, r1, bl1.reshape(1, H), Wl2, Wr2)
    (P2,) = _sc_segsum(g2, src_g2, dst_g2, zeros64, zeros16, ones128,
                       with_cnt=False)
    A, B = _tc_conv_combine(P2, CNT, r2, bl2.reshape(1, H), W1a, W1b)
    T = _sc_edge_gather(jnp.stack([A, B]), SD)
    return _tc_score(T, edge_attr, Wf, cvec, vv.reshape(1, H), c0).reshape(-1, 1)


# speedup vs baseline: 1.4604x; 1.4604x over previous
import functools

import jax
import jax.numpy as jnp
from jax import lax
from jax.experimental import pallas as pl
from jax.experimental.pallas import tpu as pltpu
from jax.experimental.pallas import tpu_sc as plsc

_F32 = jnp.float32
_NC = 2
_NS = 16
_NW = _NC * _NS
_CH = 128
_SC_PARAMS = pltpu.CompilerParams(use_tc_tiling_on_sc=False)


def _sc_mesh():
    return plsc.VectorSubcoreMesh(core_axis_name="c", subcore_axis_name="s",
                                  num_cores=_NC, num_subcores=_NS)


def _sc_segsum(g, src_g, dst_g, zeros64, zeros16, ones128, with_cnt):
    NP, H = g.shape
    ngr, G, _ = src_g.shape
    nsr = NP // _NS

    stage = False
    out_type = [jax.ShapeDtypeStruct((2 * NP, H), _F32)]
    scratch = [
        pltpu.VMEM((G, _CH), jnp.int32),
        pltpu.VMEM((G, _CH), jnp.int32),
        pltpu.VMEM((G * _CH, H), _F32),
        pltpu.VMEM_SHARED((NP, H), _F32),
        pltpu.SemaphoreType.DMA,
    ]
    if stage:
        scratch.append(pltpu.VMEM_SHARED((NP, H), _F32))
    if with_cnt:
        out_type.append(jax.ShapeDtypeStruct((2 * NP, 16), _F32))
        scratch += [
            pltpu.VMEM((_CH, 16), _F32),
            pltpu.VMEM_SHARED((NP, 16), _F32),
        ]

    @functools.partial(pl.kernel, out_type=out_type, mesh=_sc_mesh(),
                       scratch_types=scratch, compiler_params=_SC_PARAMS)
    def k(*refs):
        if with_cnt:
            (g_hbm, src_hbm, dst_hbm, z64_hbm, z16_hbm, ones_hbm,
             p_hbm, cnt_hbm, srcv, dstv, rows, acc, sem, onesv, cacc) = refs
            gtab = g_hbm
        elif stage:
            (g_hbm, src_hbm, dst_hbm, z64_hbm,
             p_hbm, srcv, dstv, rows, acc, sem, gsh) = refs
            gtab = gsh
        else:
            (g_hbm, src_hbm, dst_hbm, z64_hbm,
             p_hbm, srcv, dstv, rows, acc, sem) = refs
            gtab = g_hbm
        c = lax.axis_index("c")
        s = lax.axis_index("s")
        wid = s * _NC + c
        pltpu.sync_copy(z64_hbm.at[pl.ds(s * nsr, nsr)],
                        acc.at[pl.ds(s * nsr, nsr)])
        if stage:
            pltpu.sync_copy(g_hbm.at[pl.ds(s * nsr, nsr)],
                            gsh.at[pl.ds(s * nsr, nsr)])
        if with_cnt:
            pltpu.sync_copy(z16_hbm.at[pl.ds(s * nsr, nsr)],
                            cacc.at[pl.ds(s * nsr, nsr)])
            pltpu.sync_copy(ones_hbm, onesv)
        plsc.subcore_barrier()

        @pl.loop(wid, ngr, step=_NW)
        def _(grp):
            pltpu.sync_copy(src_hbm.at[grp], srcv)
            pltpu.sync_copy(dst_hbm.at[grp], dstv)
            gathers = [
                pltpu.async_copy(gtab.at[srcv.at[j]],
                                 rows.at[pl.ds(j * _CH, _CH)], sem)
                for j in range(G)
            ]
            for h in gathers:
                h.wait()
            adds = [
                pltpu.async_copy(rows.at[pl.ds(j * _CH, _CH)],
                                 acc.at[dstv.at[j]], sem, add=True)
                for j in range(G)
            ]
            if with_cnt:
                adds += [
                    pltpu.async_copy(onesv, cacc.at[dstv.at[j]], sem,
                                     add=True)
                    for j in range(G)
                ]
            for h in adds:
                h.wait()

        plsc.subcore_barrier()
        pltpu.sync_copy(acc.at[pl.ds(s * nsr, nsr)],
                        p_hbm.at[pl.ds(c * NP + s * nsr, nsr)])
        if with_cnt:
            pltpu.sync_copy(cacc.at[pl.ds(s * nsr, nsr)],
                            cnt_hbm.at[pl.ds(c * NP + s * nsr, nsr)])

    if with_cnt:
        return k(g, src_g, dst_g, zeros64, zeros16, ones128)
    return k(g, src_g, dst_g, zeros64)


def _sc_edge_gather(AB2, SD):
    _, NP, H = AB2.shape
    _, ngr, G, _ = SD.shape
    E = ngr * G * _CH
    slab = G * _CH

    @functools.partial(
        pl.kernel,
        out_type=jax.ShapeDtypeStruct((E, 2 * H), _F32),
        mesh=_sc_mesh(),
        compiler_params=_SC_PARAMS,
        scratch_types=[
            pltpu.VMEM((G, _CH), jnp.int32),
            pltpu.VMEM((slab, H), _F32),
            pltpu.VMEM_SHARED((NP, H), _F32),
            pltpu.SemaphoreType.DMA,
        ],
    )
    def k(ab_hbm, sd_hbm, t_hbm, idxv, rows, tsh, sem):
        c = lax.axis_index("c")
        s = lax.axis_index("s")
        nsr = NP // _NS
        pltpu.sync_copy(ab_hbm.at[c, pl.ds(s * nsr, nsr)],
                        tsh.at[pl.ds(s * nsr, nsr)])
        plsc.subcore_barrier()

        @pl.loop(s, ngr, step=_NS)
        def _(grp):
            pltpu.sync_copy(sd_hbm.at[c, grp], idxv)
            gathers = [
                pltpu.async_copy(tsh.at[idxv.at[j]],
                                 rows.at[pl.ds(j * _CH, _CH)], sem)
                for j in range(G)
            ]
            for h in gathers:
                h.wait()
            pltpu.sync_copy(rows, t_hbm.at[pl.ds(grp * slab, slab),
                                           pl.ds(c * H, H)])

    return k(AB2, SD)


def _tc_stage1(x, Wn, bn, Wl1, Wr1, NP):
    N, D = x.shape
    H = Wn.shape[1]
    BN = 2000

    def body(x_ref, wn_ref, bn_ref, wl_ref, wr_ref, g_ref, r_ref):
        h0 = jnp.maximum(x_ref[...] @ wn_ref[...] + bn_ref[...], 0.0)
        g_ref[...] = h0 @ wl_ref[...]
        r_ref[...] = h0 @ wr_ref[...]

    return pl.pallas_call(
        body,
        grid=(N // BN,),
        in_specs=[
            pl.BlockSpec((BN, D), lambda i: (i, 0)),
            pl.BlockSpec((D, H), lambda i: (0, 0)),
            pl.BlockSpec((1, H), lambda i: (0, 0)),
            pl.BlockSpec((H, H), lambda i: (0, 0)),
            pl.BlockSpec((H, H), lambda i: (0, 0)),
        ],
        out_specs=[pl.BlockSpec((BN, H), lambda i: (i, 0))] * 2,
        out_shape=[jax.ShapeDtypeStruct((NP, H), _F32)] * 2,
    )(x, Wn, bn, Wl1, Wr1)


def _tc_conv_combine(P, CNT, r, bl, Wl, Wr):
    twoNP, H = P.shape
    NP = twoNP // 2
    BN = NP // 8
    NB = NP // BN

    def body(p0, p1, c0r, c1r, r_ref, bl_ref, wl_ref, wr_ref, g_ref, r2_ref):
        S = p0[...] + p1[...]
        cnt = c0r[...][:, 0:1] + c1r[...][:, 0:1]
        h = jnp.maximum(S / jnp.maximum(cnt, 1.0) + bl_ref[...] + r_ref[...],
                        0.0)
        g_ref[...] = h @ wl_ref[...]
        r2_ref[...] = h @ wr_ref[...]

    return pl.pallas_call(
        body,
        grid=(NB,),
        in_specs=[
            pl.BlockSpec((BN, H), lambda i: (i, 0)),
            pl.BlockSpec((BN, H), lambda i: (i + NB, 0)),
            pl.BlockSpec((BN, 16), lambda i: (i, 0)),
            pl.BlockSpec((BN, 16), lambda i: (i + NB, 0)),
            pl.BlockSpec((BN, H), lambda i: (i, 0)),
            pl.BlockSpec((1, H), lambda i: (0, 0)),
            pl.BlockSpec((H, H), lambda i: (0, 0)),
            pl.BlockSpec((H, H), lambda i: (0, 0)),
        ],
        out_specs=[pl.BlockSpec((BN, H), lambda i: (i, 0))] * 2,
        out_shape=[jax.ShapeDtypeStruct((NP, H), _F32)] * 2,
    )(P, P, CNT, CNT, r, bl, Wl, Wr)


def _tc_score(T, ea, Wf, cvec, vv, c0):
    E, H2 = T.shape
    H = H2 // 2
    De = ea.shape[1]
    BE = 3200
    NB = E // BE

    def body(t_ref, ea_ref, wf_ref, cv_ref, v_ref, c0_ref, o_ref):
        i = pl.program_id(0)
        tt = t_ref[...]
        t = tt[:, :H] + tt[:, H:] + ea_ref[...] @ wf_ref[...] + cv_ref[...]
        sc = (jnp.sum(jnp.maximum(t, 0.0) * v_ref[...], axis=1)
              + c0_ref[0, 0])
        o_ref[pl.ds(i, 1), :] = sc.reshape(1, BE)

    return pl.pallas_call(
        body,
        grid=(NB,),
        in_specs=[
            pl.BlockSpec((BE, 2 * H), lambda i: (i, 0)),
            pl.BlockSpec((BE, De), lambda i: (i, 0)),
            pl.BlockSpec((De, H), lambda i: (0, 0)),
            pl.BlockSpec((1, H), lambda i: (0, 0)),
            pl.BlockSpec((1, H), lambda i: (0, 0)),
            pl.BlockSpec((1, 1), lambda i: (0, 0)),
        ],
        out_specs=pl.BlockSpec((NB, BE), lambda i: (0, 0)),
        out_shape=jax.ShapeDtypeStruct((NB, BE), _F32),
    )(T, ea, Wf, cvec, vv, c0)


def kernel(x, edge_index, edge_attr, Wn, bn, Wl1, bl1, Wr1, Wl2, bl2, Wr2,
           We, be, W1, b1, W2, b2, Ws, bs):
    N = x.shape[0]
    H = Wn.shape[1]
    NP = ((N + 127) // 128) * 128

    G1, G2, GE = 5, 10, 5
    src = edge_index[0]
    dst = edge_index[1]
    src_g1 = src.reshape(-1, G1, _CH)
    dst_g1 = dst.reshape(-1, G1, _CH)
    src_g2 = src.reshape(-1, G2, _CH)
    dst_g2 = dst.reshape(-1, G2, _CH)
    SD = jnp.stack([src, dst]).reshape(2, -1, GE, _CH)
    zeros64 = jnp.zeros((NP, H), _F32)
    zeros16 = jnp.zeros((NP, 16), _F32)
    ones128 = jnp.ones((_CH, 16), _F32)

    W1a, W1b, W1c = W1[:H], W1[H:2 * H], W1[2 * H:]
    Wf = We @ W1c
    cvec = (be @ W1c + b1).reshape(1, H)
    vv = W2 @ Ws
    c0 = (b2 @ Ws + bs).reshape(1, 1)

    g1, r1 = _tc_stage1(x, Wn, bn.reshape(1, H), Wl1, Wr1, NP)
    P1, CNT = _sc_segsum(g1, src_g1, dst_g1, zeros64, zeros16, ones128,
                         with_cnt=True)
    g2, r2 = _tc_conv_combine(P1, CNT, r1, bl1.reshape(1, H), Wl2, Wr2)
    (P2,) = _sc_segsum(g2, src_g2, dst_g2, zeros64, zeros16, ones128,
                       with_cnt=False)
    A, B = _tc_conv_combine(P2, CNT, r2, bl2.reshape(1, H), W1a, W1b)
    T = _sc_edge_gather(jnp.stack([A, B]), SD)
    return _tc_score(T, edge_attr, Wf, cvec, vv.reshape(1, H), c0).reshape(-1, 1)


# speedup vs baseline: 1.5453x; 1.0582x over previous
import functools

import jax
import jax.numpy as jnp
from jax import lax
from jax.experimental import pallas as pl
from jax.experimental.pallas import tpu as pltpu
from jax.experimental.pallas import tpu_sc as plsc

_F32 = jnp.float32
_NC = 2
_NS = 16
_NW = _NC * _NS
_CH = 128
_SC_PARAMS = pltpu.CompilerParams(use_tc_tiling_on_sc=False)


def _sc_mesh():
    return plsc.VectorSubcoreMesh(core_axis_name="c", subcore_axis_name="s",
                                  num_cores=_NC, num_subcores=_NS)


def _sc_segsum(g, src_g, dst_g, zeros64, zeros16, ones128, with_cnt):
    NP, H = g.shape
    ngr, G, _ = src_g.shape
    nsr = NP // _NS

    stage = False
    out_type = [jax.ShapeDtypeStruct((2 * NP, H), _F32)]
    scratch = [
        pltpu.VMEM((2, G, _CH), jnp.int32),
        pltpu.VMEM((2, G, _CH), jnp.int32),
        pltpu.VMEM((2, G * _CH, H), _F32),
        pltpu.VMEM_SHARED((NP, H), _F32),
        pltpu.SemaphoreType.DMA,
        pltpu.SemaphoreType.DMA,
    ]
    if stage:
        scratch.append(pltpu.VMEM_SHARED((NP, H), _F32))
    if with_cnt:
        out_type.append(jax.ShapeDtypeStruct((2 * NP, 16), _F32))
        scratch += [
            pltpu.VMEM((_CH, 16), _F32),
            pltpu.VMEM_SHARED((NP, 16), _F32),
        ]

    @functools.partial(pl.kernel, out_type=out_type, mesh=_sc_mesh(),
                       scratch_types=scratch, compiler_params=_SC_PARAMS)
    def k(*refs):
        if with_cnt:
            (g_hbm, src_hbm, dst_hbm, z64_hbm, z16_hbm, ones_hbm,
             p_hbm, cnt_hbm, srcv, dstv, rows, acc, semg, sema,
             onesv, cacc) = refs
            gtab = g_hbm
        elif stage:
            (g_hbm, src_hbm, dst_hbm, z64_hbm,
             p_hbm, srcv, dstv, rows, acc, semg, sema, gsh) = refs
            gtab = gsh
        else:
            (g_hbm, src_hbm, dst_hbm, z64_hbm,
             p_hbm, srcv, dstv, rows, acc, semg, sema) = refs
            gtab = g_hbm
        c = lax.axis_index("c")
        s = lax.axis_index("s")
        wid = s * _NC + c
        pltpu.sync_copy(z64_hbm.at[pl.ds(s * nsr, nsr)],
                        acc.at[pl.ds(s * nsr, nsr)])
        if stage:
            pltpu.sync_copy(g_hbm.at[pl.ds(s * nsr, nsr)],
                            gsh.at[pl.ds(s * nsr, nsr)])
        if with_cnt:
            pltpu.sync_copy(z16_hbm.at[pl.ds(s * nsr, nsr)],
                            cacc.at[pl.ds(s * nsr, nsr)])
            pltpu.sync_copy(ones_hbm, onesv)
        plsc.subcore_barrier()

        def load_idx(grp, b):
            pltpu.sync_copy(src_hbm.at[grp], srcv.at[b])
            pltpu.sync_copy(dst_hbm.at[grp], dstv.at[b])

        def fire_gathers(b):
            return [
                pltpu.async_copy(gtab.at[srcv.at[b, j]],
                                 rows.at[b, pl.ds(j * _CH, _CH)], semg)
                for j in range(G)
            ]

        def fire_adds(b):
            adds = [
                pltpu.async_copy(rows.at[b, pl.ds(j * _CH, _CH)],
                                 acc.at[dstv.at[b, j]], sema, add=True)
                for j in range(G)
            ]
            if with_cnt:
                adds += [
                    pltpu.async_copy(onesv, cacc.at[dstv.at[b, j]], sema,
                                     add=True)
                    for j in range(G)
                ]
            return adds

        @pl.loop(wid, ngr, step=2 * _NW)
        def _(grp0):
            grp1 = grp0 + _NW
            load_idx(grp0, 0)
            ga = fire_gathers(0)

            @pl.when(grp1 < ngr)
            def _():
                load_idx(grp1, 1)

            for h in ga:
                h.wait()
            aa = fire_adds(0)

            @pl.when(grp1 < ngr)
            def _():
                gb = fire_gathers(1)
                for h in gb:
                    h.wait()

            for h in aa:
                h.wait()

            @pl.when(grp1 < ngr)
            def _():
                ab = fire_adds(1)
                for h in ab:
                    h.wait()

        plsc.subcore_barrier()
        pltpu.sync_copy(acc.at[pl.ds(s * nsr, nsr)],
                        p_hbm.at[pl.ds(c * NP + s * nsr, nsr)])
        if with_cnt:
            pltpu.sync_copy(cacc.at[pl.ds(s * nsr, nsr)],
                            cnt_hbm.at[pl.ds(c * NP + s * nsr, nsr)])

    if with_cnt:
        return k(g, src_g, dst_g, zeros64, zeros16, ones128)
    return k(g, src_g, dst_g, zeros64)


def _sc_edge_gather(AB2, SD):
    _, NP, H = AB2.shape
    _, ngr, G, _ = SD.shape
    E = ngr * G * _CH
    slab = G * _CH

    @functools.partial(
        pl.kernel,
        out_type=jax.ShapeDtypeStruct((E, 2 * H), _F32),
        mesh=_sc_mesh(),
        compiler_params=_SC_PARAMS,
        scratch_types=[
            pltpu.VMEM((2, G, _CH), jnp.int32),
            pltpu.VMEM((2, slab, H), _F32),
            pltpu.VMEM_SHARED((NP, H), _F32),
            pltpu.SemaphoreType.DMA,
            pltpu.SemaphoreType.DMA,
        ],
    )
    def k(ab_hbm, sd_hbm, t_hbm, idxv, rows, tsh, semg, semw):
        c = lax.axis_index("c")
        s = lax.axis_index("s")
        nsr = NP // _NS
        pltpu.sync_copy(ab_hbm.at[c, pl.ds(s * nsr, nsr)],
                        tsh.at[pl.ds(s * nsr, nsr)])
        plsc.subcore_barrier()

        def fire_gathers(b):
            return [
                pltpu.async_copy(tsh.at[idxv.at[b, j]],
                                 rows.at[b, pl.ds(j * _CH, _CH)], semg)
                for j in range(G)
            ]

        def fire_write(grp, b):
            return pltpu.async_copy(
                rows.at[b], t_hbm.at[pl.ds(grp * slab, slab),
                                     pl.ds(c * H, H)], semw)

        @pl.loop(s, ngr, step=2 * _NS)
        def _(grp0):
            grp1 = grp0 + _NS
            pltpu.sync_copy(sd_hbm.at[c, grp0], idxv.at[0])
            ga = fire_gathers(0)

            @pl.when(grp1 < ngr)
            def _():
                pltpu.sync_copy(sd_hbm.at[c, grp1], idxv.at[1])

            for h in ga:
                h.wait()
            wa = fire_write(grp0, 0)

            @pl.when(grp1 < ngr)
            def _():
                gb = fire_gathers(1)
                for h in gb:
                    h.wait()

            wa.wait()

            @pl.when(grp1 < ngr)
            def _():
                fire_write(grp1, 1).wait()

    return k(AB2, SD)


def _tc_stage1(x, Wn, bn, Wl1, Wr1, NP):
    N, D = x.shape
    H = Wn.shape[1]
    BN = 2000

    def body(x_ref, wn_ref, bn_ref, wl_ref, wr_ref, g_ref, r_ref):
        h0 = jnp.maximum(x_ref[...] @ wn_ref[...] + bn_ref[...], 0.0)
        g_ref[...] = h0 @ wl_ref[...]
        r_ref[...] = h0 @ wr_ref[...]

    return pl.pallas_call(
        body,
        grid=(N // BN,),
        in_specs=[
            pl.BlockSpec((BN, D), lambda i: (i, 0)),
            pl.BlockSpec((D, H), lambda i: (0, 0)),
            pl.BlockSpec((1, H), lambda i: (0, 0)),
            pl.BlockSpec((H, H), lambda i: (0, 0)),
            pl.BlockSpec((H, H), lambda i: (0, 0)),
        ],
        out_specs=[pl.BlockSpec((BN, H), lambda i: (i, 0))] * 2,
        out_shape=[jax.ShapeDtypeStruct((NP, H), _F32)] * 2,
    )(x, Wn, bn, Wl1, Wr1)


def _tc_conv_combine(P, CNT, r, bl, Wl, Wr):
    twoNP, H = P.shape
    NP = twoNP // 2
    BN = NP // 8
    NB = NP // BN

    def body(p0, p1, c0r, c1r, r_ref, bl_ref, wl_ref, wr_ref, g_ref, r2_ref):
        S = p0[...] + p1[...]
        cnt = c0r[...][:, 0:1] + c1r[...][:, 0:1]
        h = jnp.maximum(S / jnp.maximum(cnt, 1.0) + bl_ref[...] + r_ref[...],
                        0.0)
        g_ref[...] = h @ wl_ref[...]
        r2_ref[...] = h @ wr_ref[...]

    return pl.pallas_call(
        body,
        grid=(NB,),
        in_specs=[
            pl.BlockSpec((BN, H), lambda i: (i, 0)),
            pl.BlockSpec((BN, H), lambda i: (i + NB, 0)),
            pl.BlockSpec((BN, 16), lambda i: (i, 0)),
            pl.BlockSpec((BN, 16), lambda i: (i + NB, 0)),
            pl.BlockSpec((BN, H), lambda i: (i, 0)),
            pl.BlockSpec((1, H), lambda i: (0, 0)),
            pl.BlockSpec((H, H), lambda i: (0, 0)),
            pl.BlockSpec((H, H), lambda i: (0, 0)),
        ],
        out_specs=[pl.BlockSpec((BN, H), lambda i: (i, 0))] * 2,
        out_shape=[jax.ShapeDtypeStruct((NP, H), _F32)] * 2,
    )(P, P, CNT, CNT, r, bl, Wl, Wr)


def _tc_score(T, ea, Wf, cvec, vv, c0):
    E, H2 = T.shape
    H = H2 // 2
    De = ea.shape[1]
    BE = 3200
    NB = E // BE

    def body(t_ref, ea_ref, wf_ref, cv_ref, v_ref, c0_ref, o_ref):
        i = pl.program_id(0)
        tt = t_ref[...]
        t = tt[:, :H] + tt[:, H:] + ea_ref[...] @ wf_ref[...] + cv_ref[...]
        sc = (jnp.sum(jnp.maximum(t, 0.0) * v_ref[...], axis=1)
              + c0_ref[0, 0])
        o_ref[pl.ds(i, 1), :] = sc.reshape(1, BE)

    return pl.pallas_call(
        body,
        grid=(NB,),
        in_specs=[
            pl.BlockSpec((BE, 2 * H), lambda i: (i, 0)),
            pl.BlockSpec((BE, De), lambda i: (i, 0)),
            pl.BlockSpec((De, H), lambda i: (0, 0)),
            pl.BlockSpec((1, H), lambda i: (0, 0)),
            pl.BlockSpec((1, H), lambda i: (0, 0)),
            pl.BlockSpec((1, 1), lambda i: (0, 0)),
        ],
        out_specs=pl.BlockSpec((NB, BE), lambda i: (0, 0)),
        out_shape=jax.ShapeDtypeStruct((NB, BE), _F32),
    )(T, ea, Wf, cvec, vv, c0)


def kernel(x, edge_index, edge_attr, Wn, bn, Wl1, bl1, Wr1, Wl2, bl2, Wr2,
           We, be, W1, b1, W2, b2, Ws, bs):
    N = x.shape[0]
    H = Wn.shape[1]
    NP = ((N + 127) // 128) * 128

    G1, G2, GE = 4, 5, 5
    src = edge_index[0]
    dst = edge_index[1]
    src_g1 = src.reshape(-1, G1, _CH)
    dst_g1 = dst.reshape(-1, G1, _CH)
    src_g2 = src.reshape(-1, G2, _CH)
    dst_g2 = dst.reshape(-1, G2, _CH)
    SD = jnp.stack([src, dst]).reshape(2, -1, GE, _CH)
    zeros64 = jnp.zeros((NP, H), _F32)
    zeros16 = jnp.zeros((NP, 16), _F32)
    ones128 = jnp.ones((_CH, 16), _F32)

    W1a, W1b, W1c = W1[:H], W1[H:2 * H], W1[2 * H:]
    Wf = We @ W1c
    cvec = (be @ W1c + b1).reshape(1, H)
    vv = W2 @ Ws
    c0 = (b2 @ Ws + bs).reshape(1, 1)

    g1, r1 = _tc_stage1(x, Wn, bn.reshape(1, H), Wl1, Wr1, NP)
    P1, CNT = _sc_segsum(g1, src_g1, dst_g1, zeros64, zeros16, ones128,
                         with_cnt=True)
    g2, r2 = _tc_conv_combine(P1, CNT, r1, bl1.reshape(1, H), Wl2, Wr2)
    (P2,) = _sc_segsum(g2, src_g2, dst_g2, zeros64, zeros16, ones128,
                       with_cnt=False)
    A, B = _tc_conv_combine(P2, CNT, r2, bl2.reshape(1, H), W1a, W1b)
    T = _sc_edge_gather(jnp.stack([A, B]), SD)
    return _tc_score(T, edge_attr, Wf, cvec, vv.reshape(1, H), c0).reshape(-1, 1)
